# initial kernel scaffold (unmeasured)
import jax
import jax.numpy as jnp
from jax import lax
from jax.experimental import pallas as pl
from jax.experimental.pallas import tpu as pltpu

N_DEV = 16


def kernel(A, B):
    m_per, k = A.shape
    _, n = B.shape
    m_total = N_DEV * m_per

    def body(a_ref, b_ref, out_ref, comm_ref, send_sems, recv_sems, copy_sem):
        my_pos = lax.axis_index("i")
        left = lax.rem(my_pos - 1 + N_DEV, N_DEV)
        right = lax.rem(my_pos + 1, N_DEV)

        barrier_sem = pltpu.get_barrier_semaphore()
        for nbr in (left, right):
            pl.semaphore_signal(
                barrier_sem, inc=1,
                device_id=(nbr,), device_id_type=pl.DeviceIdType.MESH,
            )
        pl.semaphore_wait(barrier_sem, 2)

        a = a_ref[...].astype(jnp.bfloat16)
        b = b_ref[...].astype(jnp.bfloat16)
        c = jnp.dot(a, b, preferred_element_type=jnp.float32).astype(jnp.bfloat16)
        comm_ref[0, :, :] = c

        cp = pltpu.make_async_copy(
            comm_ref.at[0], out_ref.at[pl.ds(my_pos * m_per, m_per), :], copy_sem
        )
        cp.start()
        cp.wait()

        for h in range(N_DEV - 1):
            send_slot = h % 2
            recv_slot = (h + 1) % 2
            rdma = pltpu.make_async_remote_copy(
                src_ref=comm_ref.at[send_slot],
                dst_ref=comm_ref.at[recv_slot],
                send_sem=send_sems.at[send_slot],
                recv_sem=recv_sems.at[recv_slot],
                device_id=(right,),
                device_id_type=pl.DeviceIdType.MESH,
            )
            rdma.start()
            rdma.wait()

            origin = lax.rem(my_pos - (h + 1) + N_DEV, N_DEV)
            cp = pltpu.make_async_copy(
                comm_ref.at[recv_slot],
                out_ref.at[pl.ds(origin * m_per, m_per), :],
                copy_sem,
            )
            cp.start()
            cp.wait()

    return pl.pallas_call(
        body,
        out_shape=jax.ShapeDtypeStruct((m_total, n), jnp.bfloat16),
        in_specs=[
            pl.BlockSpec(memory_space=pltpu.VMEM),
            pl.BlockSpec(memory_space=pltpu.VMEM),
        ],
        out_specs=pl.BlockSpec(memory_space=pltpu.ANY),
        scratch_shapes=[
            pltpu.VMEM((2, m_per, n), jnp.bfloat16),
            pltpu.SemaphoreType.DMA((2,)),
            pltpu.SemaphoreType.DMA((2,)),
            pltpu.SemaphoreType.DMA,
        ],
        compiler_params=pltpu.CompilerParams(collective_id=0),
    )(A, B)


# baseline (device time: 1533081 ns/iter reference)
import jax
import jax.numpy as jnp
from jax import lax
from jax.experimental import pallas as pl
from jax.experimental.pallas import tpu as pltpu

N_DEV = 16


def kernel(A, B):
    m_per, k = A.shape
    _, n = B.shape
    m_total = N_DEV * m_per

    def body(a_ref, b_ref, out_ref, comm_ref, send_sems, recv_sems, copy_sem):
        my_pos = lax.axis_index("i")
        left = lax.rem(my_pos - 1 + N_DEV, N_DEV)
        right = lax.rem(my_pos + 1, N_DEV)

        barrier_sem = pltpu.get_barrier_semaphore()
        for nbr in (left, right):
            pl.semaphore_signal(
                barrier_sem, inc=1,
                device_id=(nbr,), device_id_type=pl.DeviceIdType.MESH,
            )
        pl.semaphore_wait(barrier_sem, 2)

        a = a_ref[...].astype(jnp.bfloat16)
        b = b_ref[...].astype(jnp.bfloat16)
        c = jnp.dot(a, b, preferred_element_type=jnp.float32).astype(jnp.bfloat16)
        comm_ref[0, :, :] = c

        cp = pltpu.make_async_copy(
            comm_ref.at[0], out_ref.at[pl.ds(my_pos * m_per, m_per), :], copy_sem
        )
        cp.start()
        cp.wait()

        for h in range(N_DEV - 1):
            send_slot = h % 2
            recv_slot = (h + 1) % 2
            rdma = pltpu.make_async_remote_copy(
                src_ref=comm_ref.at[send_slot],
                dst_ref=comm_ref.at[recv_slot],
                send_sem=send_sems.at[send_slot],
                recv_sem=recv_sems.at[recv_slot],
                device_id=(right,),
                device_id_type=pl.DeviceIdType.MESH,
            )
            rdma.start()
            rdma.wait()

            origin = lax.rem(my_pos - (h + 1) + N_DEV, N_DEV)
            cp = pltpu.make_async_copy(
                comm_ref.at[recv_slot],
                out_ref.at[pl.ds(origin * m_per, m_per), :],
                copy_sem,
            )
            cp.start()
            cp.wait()

    return pl.pallas_call(
        body,
        out_shape=jax.ShapeDtypeStruct((m_total, n), jnp.bfloat16),
        in_specs=[
            pl.BlockSpec(memory_space=pltpu.VMEM),
            pl.BlockSpec(memory_space=pltpu.VMEM),
        ],
        out_specs=pl.BlockSpec(memory_space=pl.ANY),
        scratch_shapes=[
            pltpu.VMEM((2, m_per, n), jnp.bfloat16),
            pltpu.SemaphoreType.DMA((2,)),
            pltpu.SemaphoreType.DMA((2,)),
            pltpu.SemaphoreType.DMA,
        ],
        compiler_params=pltpu.CompilerParams(collective_id=0),
    )(A, B)


# device time: 850420 ns/iter; 1.8027x vs baseline; 1.8027x over previous
import jax
import jax.numpy as jnp
from jax import lax
from jax.experimental import pallas as pl
from jax.experimental.pallas import tpu as pltpu

N_DEV = 16
CW_HOPS = N_DEV // 2
CCW_HOPS = N_DEV - 1 - CW_HOPS


def kernel(A, B):
    m_per, k = A.shape
    _, n = B.shape
    m_total = N_DEV * m_per

    def body(a_ref, b_ref, out_ref, cw_ref, ccw_ref,
             cw_send, cw_recv, ccw_send, ccw_recv, copy_sems, own_sem):
        my_pos = lax.axis_index("i")
        left = lax.rem(my_pos - 1 + N_DEV, N_DEV)
        right = lax.rem(my_pos + 1, N_DEV)

        barrier_sem = pltpu.get_barrier_semaphore()
        for nbr in (left, right):
            pl.semaphore_signal(
                barrier_sem, inc=1,
                device_id=(nbr,), device_id_type=pl.DeviceIdType.MESH,
            )

        a = a_ref[...].astype(jnp.bfloat16)
        b = b_ref[...].astype(jnp.bfloat16)
        c = jnp.dot(a, b, preferred_element_type=jnp.float32).astype(jnp.bfloat16)
        cw_ref[0, :, :] = c
        ccw_ref[0, :, :] = c

        own_cp = pltpu.make_async_copy(
            cw_ref.at[0], out_ref.at[pl.ds(my_pos * m_per, m_per), :], own_sem
        )
        own_cp.start()

        pl.semaphore_wait(barrier_sem, 2)

        def hbm_copy(src_slot_ref, origin, sem):
            cp = pltpu.make_async_copy(
                src_slot_ref, out_ref.at[pl.ds(origin * m_per, m_per), :], sem
            )
            cp.start()
            return cp

        pending = []
        for h in range(CW_HOPS):
            s, r = h % 2, (h + 1) % 2
            cw = pltpu.make_async_remote_copy(
                src_ref=cw_ref.at[s], dst_ref=cw_ref.at[r],
                send_sem=cw_send.at[s], recv_sem=cw_recv.at[r],
                device_id=(right,), device_id_type=pl.DeviceIdType.MESH,
            )
            cw.start()
            if h < CCW_HOPS:
                ccw = pltpu.make_async_remote_copy(
                    src_ref=ccw_ref.at[s], dst_ref=ccw_ref.at[r],
                    send_sem=ccw_send.at[s], recv_sem=ccw_recv.at[r],
                    device_id=(left,), device_id_type=pl.DeviceIdType.MESH,
                )
                ccw.start()

            for cp in pending:
                cp.wait()
            pending = []
            if h == 0:
                own_cp.wait()

            cw.wait()
            origin_cw = lax.rem(my_pos - (h + 1) + N_DEV, N_DEV)
            pending.append(hbm_copy(cw_ref.at[r], origin_cw, copy_sems.at[0]))
            if h < CCW_HOPS:
                ccw.wait()
                origin_ccw = lax.rem(my_pos + (h + 1), N_DEV)
                pending.append(hbm_copy(ccw_ref.at[r], origin_ccw, copy_sems.at[1]))

        for cp in pending:
            cp.wait()

    return pl.pallas_call(
        body,
        out_shape=jax.ShapeDtypeStruct((m_total, n), jnp.bfloat16),
        in_specs=[
            pl.BlockSpec(memory_space=pltpu.VMEM),
            pl.BlockSpec(memory_space=pltpu.VMEM),
        ],
        out_specs=pl.BlockSpec(memory_space=pl.ANY),
        scratch_shapes=[
            pltpu.VMEM((2, m_per, n), jnp.bfloat16),
            pltpu.VMEM((2, m_per, n), jnp.bfloat16),
            pltpu.SemaphoreType.DMA((2,)),
            pltpu.SemaphoreType.DMA((2,)),
            pltpu.SemaphoreType.DMA((2,)),
            pltpu.SemaphoreType.DMA((2,)),
            pltpu.SemaphoreType.DMA((2,)),
            pltpu.SemaphoreType.DMA,
        ],
        compiler_params=pltpu.CompilerParams(
            collective_id=0, vmem_limit_bytes=100 * 1024 * 1024
        ),
    )(A, B)


# device time: 785409 ns/iter; 1.9520x vs baseline; 1.0828x over previous
import jax
import jax.numpy as jnp
from jax import lax
from jax.experimental import pallas as pl
from jax.experimental.pallas import tpu as pltpu

N_DEV = 16
HOPS = N_DEV // 2


def kernel(A, B):
    m_per, k = A.shape
    _, n = B.shape
    m_total = N_DEV * m_per
    half = m_per // 2

    def body(a_ref, b_ref, out_ref, cw_ref, ccw_ref,
             cw_send, cw_recv, ccw_send, ccw_recv, copy_sems, own_sem):
        my_pos = lax.axis_index("i")
        left = lax.rem(my_pos - 1 + N_DEV, N_DEV)
        right = lax.rem(my_pos + 1, N_DEV)

        def sub(ref, slot, j):
            return ref.at[slot, pl.ds(j * half, half), :]

        def mk(direction, h, j):
            s, r = h % 2, (h + 1) % 2
            if direction == "cw":
                ref, send, recv, tgt = cw_ref, cw_send, cw_recv, right
            else:
                ref, send, recv, tgt = ccw_ref, ccw_send, ccw_recv, left
            return pltpu.make_async_remote_copy(
                src_ref=sub(ref, s, j), dst_ref=sub(ref, r, j),
                send_sem=send.at[s, j], recv_sem=recv.at[r, j],
                device_id=(tgt,), device_id_type=pl.DeviceIdType.MESH,
            )

        def subs(direction, h):
            if h < HOPS - 1:
                return (0, 1)
            return (0,) if direction == "cw" else (1,)

        barrier_sem = pltpu.get_barrier_semaphore()
        for nbr in (left, right):
            pl.semaphore_signal(
                barrier_sem, inc=1,
                device_id=(nbr,), device_id_type=pl.DeviceIdType.MESH,
            )

        a = a_ref[...].astype(jnp.bfloat16)
        b = b_ref[...].astype(jnp.bfloat16)
        c = jnp.dot(a, b, preferred_element_type=jnp.float32).astype(jnp.bfloat16)
        cw_ref[0, :, :] = c
        ccw_ref[0, :, :] = c

        own_cp = pltpu.make_async_copy(
            cw_ref.at[0], out_ref.at[pl.ds(my_pos * m_per, m_per), :], own_sem
        )
        own_cp.start()

        pl.semaphore_wait(barrier_sem, 2)

        rd = {}
        for d in ("cw", "ccw"):
            for j in subs(d, 0):
                rd[(d, 0, j)] = mk(d, 0, j)
                rd[(d, 0, j)].start()

        def hbm_copy(src_ref, out_row, rows, sem):
            cp = pltpu.make_async_copy(
                src_ref, out_ref.at[pl.ds(out_row, rows), :], sem
            )
            cp.start()
            return cp

        pending = []
        for h in range(1, HOPS):
            for j in (0, 1):
                for d in ("cw", "ccw"):
                    rd[(d, h - 1, j)].wait_recv()
                    if j in subs(d, h):
                        if h >= 2:
                            rd[(d, h - 2, j)].wait_send()
                        rd[(d, h, j)] = mk(d, h, j)
                        rd[(d, h, j)].start()
            for cp in pending:
                cp.wait()
            pending = []
            slot = h % 2
            origin_cw = lax.rem(my_pos - h + N_DEV, N_DEV)
            origin_ccw = lax.rem(my_pos + h, N_DEV)
            pending.append(hbm_copy(cw_ref.at[slot], origin_cw * m_per,
                                    m_per, copy_sems.at[0]))
            pending.append(hbm_copy(ccw_ref.at[slot], origin_ccw * m_per,
                                    m_per, copy_sems.at[1]))

        rd[("cw", HOPS - 1, 0)].wait_recv()
        rd[("ccw", HOPS - 1, 1)].wait_recv()
        for cp in pending:
            cp.wait()
        slot = HOPS % 2
        origin = lax.rem(my_pos + HOPS, N_DEV)
        hbm_copy(sub(cw_ref, slot, 0), origin * m_per, half,
                 copy_sems.at[0]).wait()
        hbm_copy(sub(ccw_ref, slot, 1), origin * m_per + half, half,
                 copy_sems.at[1]).wait()
        own_cp.wait()

        for d in ("cw", "ccw"):
            for j in (0, 1):
                if j not in subs(d, HOPS - 1):
                    rd[(d, HOPS - 3, j)].wait_send()
            for j in subs(d, HOPS - 2):
                rd[(d, HOPS - 2, j)].wait_send()
            for j in subs(d, HOPS - 1):
                rd[(d, HOPS - 1, j)].wait_send()

    return pl.pallas_call(
        body,
        out_shape=jax.ShapeDtypeStruct((m_total, n), jnp.bfloat16),
        in_specs=[
            pl.BlockSpec(memory_space=pltpu.VMEM),
            pl.BlockSpec(memory_space=pltpu.VMEM),
        ],
        out_specs=pl.BlockSpec(memory_space=pl.ANY),
        scratch_shapes=[
            pltpu.VMEM((2, m_per, n), jnp.bfloat16),
            pltpu.VMEM((2, m_per, n), jnp.bfloat16),
            pltpu.SemaphoreType.DMA((2, 2)),
            pltpu.SemaphoreType.DMA((2, 2)),
            pltpu.SemaphoreType.DMA((2, 2)),
            pltpu.SemaphoreType.DMA((2, 2)),
            pltpu.SemaphoreType.DMA((2,)),
            pltpu.SemaphoreType.DMA,
        ],
        compiler_params=pltpu.CompilerParams(
            collective_id=0, vmem_limit_bytes=100 * 1024 * 1024
        ),
    )(A, B)


# device time: 779283 ns/iter; 1.9673x vs baseline; 1.0079x over previous
import jax
import jax.numpy as jnp
from jax import lax
from jax.experimental import pallas as pl
from jax.experimental.pallas import tpu as pltpu

N_DEV = 16
HOPS = N_DEV // 2


def kernel(A, B):
    m_per, k = A.shape
    _, n = B.shape
    m_total = N_DEV * m_per
    half = m_per // 2

    def body(a_ref, b_ref, out_ref, c_ref,
             cw_send, cw_recv, ccw_send, ccw_recv, own_sem):
        my_pos = lax.axis_index("i")
        left = lax.rem(my_pos - 1 + N_DEV, N_DEV)
        right = lax.rem(my_pos + 1, N_DEV)

        def chunk_rows(origin, j):
            return out_ref.at[pl.ds(origin * m_per + j * half, half), :]

        def send_origin(d, h):
            if d == "cw":
                return lax.rem(my_pos - h + N_DEV, N_DEV)
            return lax.rem(my_pos + h, N_DEV)

        def sems(d):
            return (cw_send, cw_recv, right) if d == "cw" else (ccw_send, ccw_recv, left)

        def mk_out(d, h, j):
            send, recv, tgt = sems(d)
            o = send_origin(d, h)
            src = c_ref.at[pl.ds(j * half, half), :] if h == 0 else chunk_rows(o, j)
            return pltpu.make_async_remote_copy(
                src_ref=src, dst_ref=chunk_rows(o, j),
                send_sem=send.at[h % 2, j], recv_sem=recv.at[(h + 1) % 2, j],
                device_id=(tgt,), device_id_type=pl.DeviceIdType.MESH,
            )

        def mk_in(d, h, j):
            send, recv, tgt = sems(d)
            o = send_origin(d, h + 1)
            return pltpu.make_async_remote_copy(
                src_ref=c_ref.at[pl.ds(j * half, half), :], dst_ref=chunk_rows(o, j),
                send_sem=send.at[h % 2, j], recv_sem=recv.at[(h + 1) % 2, j],
                device_id=(tgt,), device_id_type=pl.DeviceIdType.MESH,
            )

        def subs(d, h):
            if h < HOPS - 1:
                return (0, 1)
            return (0,) if d == "cw" else (1,)

        barrier_sem = pltpu.get_barrier_semaphore()
        for nbr in (left, right):
            pl.semaphore_signal(
                barrier_sem, inc=1,
                device_id=(nbr,), device_id_type=pl.DeviceIdType.MESH,
            )

        a = a_ref[...].astype(jnp.bfloat16)
        b = b_ref[...].astype(jnp.bfloat16)
        c0 = jnp.dot(a[:half], b, preferred_element_type=jnp.float32)
        c_ref[pl.ds(0, half), :] = c0.astype(jnp.bfloat16)

        pl.semaphore_wait(barrier_sem, 2)

        out_rd = {}
        for d in ("cw", "ccw"):
            out_rd[(d, 0, 0)] = mk_out(d, 0, 0)
            out_rd[(d, 0, 0)].start()

        c1 = jnp.dot(a[half:], b, preferred_element_type=jnp.float32)
        c_ref[pl.ds(half, half), :] = c1.astype(jnp.bfloat16)
        for d in ("cw", "ccw"):
            out_rd[(d, 0, 1)] = mk_out(d, 0, 1)
            out_rd[(d, 0, 1)].start()

        own_cp = pltpu.make_async_copy(
            c_ref, out_ref.at[pl.ds(my_pos * m_per, m_per), :], own_sem
        )
        own_cp.start()

        for h in range(1, HOPS):
            for j in (0, 1):
                for d in ("cw", "ccw"):
                    mk_in(d, h - 1, j).wait_recv()
                    if j in subs(d, h):
                        if h >= 2:
                            out_rd[(d, h - 2, j)].wait_send()
                        out_rd[(d, h, j)] = mk_out(d, h, j)
                        out_rd[(d, h, j)].start()

        mk_in("cw", HOPS - 1, 0).wait_recv()
        mk_in("ccw", HOPS - 1, 1).wait_recv()

        for d in ("cw", "ccw"):
            for j in (0, 1):
                if j not in subs(d, HOPS - 1):
                    out_rd[(d, HOPS - 3, j)].wait_send()
            for j in subs(d, HOPS - 2):
                out_rd[(d, HOPS - 2, j)].wait_send()
            for j in subs(d, HOPS - 1):
                out_rd[(d, HOPS - 1, j)].wait_send()
        own_cp.wait()

    return pl.pallas_call(
        body,
        out_shape=jax.ShapeDtypeStruct((m_total, n), jnp.bfloat16),
        in_specs=[
            pl.BlockSpec(memory_space=pltpu.VMEM),
            pl.BlockSpec(memory_space=pltpu.VMEM),
        ],
        out_specs=pl.BlockSpec(memory_space=pl.ANY),
        scratch_shapes=[
            pltpu.VMEM((m_per, n), jnp.bfloat16),
            pltpu.SemaphoreType.DMA((2, 2)),
            pltpu.SemaphoreType.DMA((2, 2)),
            pltpu.SemaphoreType.DMA((2, 2)),
            pltpu.SemaphoreType.DMA((2, 2)),
            pltpu.SemaphoreType.DMA,
        ],
        compiler_params=pltpu.CompilerParams(
            collective_id=0, vmem_limit_bytes=100 * 1024 * 1024
        ),
    )(A, B)


# device time: 777887 ns/iter; 1.9708x vs baseline; 1.0018x over previous
import jax
import jax.numpy as jnp
from jax import lax
from jax.experimental import pallas as pl
from jax.experimental.pallas import tpu as pltpu

N_DEV = 16
HOPS = N_DEV // 2

RING = [0, 1, 5, 9, 13, 14, 10, 6, 2, 3, 7, 11, 15, 12, 8, 4]
INDEX_OF = [RING.index(p) for p in range(N_DEV)]


def kernel(A, B):
    m_per, k = A.shape
    _, n = B.shape
    m_total = N_DEV * m_per
    half = m_per // 2

    def body(a_ref, b_ref, out_ref, c_ref,
             cw_send, cw_recv, ccw_send, ccw_recv, own_sem):
        my_pos = lax.axis_index("i")

        def lookup(table, idx):
            v = jnp.int32(table[0])
            for p in range(1, N_DEV):
                v = jnp.where(idx == p, jnp.int32(table[p]), v)
            return v

        my_ring = lookup(INDEX_OF, my_pos)
        right = lookup(RING, lax.rem(my_ring + 1, N_DEV))
        left = lookup(RING, lax.rem(my_ring - 1 + N_DEV, N_DEV))

        def chunk_rows(origin, j):
            return out_ref.at[pl.ds(origin * m_per + j * half, half), :]

        def send_origin(d, h):
            if d == "cw":
                return lookup(RING, lax.rem(my_ring - h + N_DEV, N_DEV))
            return lookup(RING, lax.rem(my_ring + h, N_DEV))

        def sems(d):
            return (cw_send, cw_recv, right) if d == "cw" else (ccw_send, ccw_recv, left)

        def mk_out(d, h, j):
            send, recv, tgt = sems(d)
            o = send_origin(d, h)
            src = c_ref.at[pl.ds(j * half, half), :] if h == 0 else chunk_rows(o, j)
            return pltpu.make_async_remote_copy(
                src_ref=src, dst_ref=chunk_rows(o, j),
                send_sem=send.at[h % 2, j], recv_sem=recv.at[(h + 1) % 2, j],
                device_id=(tgt,), device_id_type=pl.DeviceIdType.MESH,
            )

        def mk_in(d, h, j):
            send, recv, tgt = sems(d)
            o = send_origin(d, h + 1)
            return pltpu.make_async_remote_copy(
                src_ref=c_ref.at[pl.ds(j * half, half), :], dst_ref=chunk_rows(o, j),
                send_sem=send.at[h % 2, j], recv_sem=recv.at[(h + 1) % 2, j],
                device_id=(tgt,), device_id_type=pl.DeviceIdType.MESH,
            )

        def subs(d, h):
            if h < HOPS - 1:
                return (0, 1)
            return (0,) if d == "cw" else (1,)

        barrier_sem = pltpu.get_barrier_semaphore()
        for nbr in (left, right):
            pl.semaphore_signal(
                barrier_sem, inc=1,
                device_id=(nbr,), device_id_type=pl.DeviceIdType.MESH,
            )

        a = a_ref[...].astype(jnp.bfloat16)
        b = b_ref[...].astype(jnp.bfloat16)
        c0 = jnp.dot(a[:half], b, preferred_element_type=jnp.float32)
        c_ref[pl.ds(0, half), :] = c0.astype(jnp.bfloat16)

        pl.semaphore_wait(barrier_sem, 2)

        out_rd = {}
        for d in ("cw", "ccw"):
            out_rd[(d, 0, 0)] = mk_out(d, 0, 0)
            out_rd[(d, 0, 0)].start()

        c1 = jnp.dot(a[half:], b, preferred_element_type=jnp.float32)
        c_ref[pl.ds(half, half), :] = c1.astype(jnp.bfloat16)
        for d in ("cw", "ccw"):
            out_rd[(d, 0, 1)] = mk_out(d, 0, 1)
            out_rd[(d, 0, 1)].start()

        own_cp = pltpu.make_async_copy(
            c_ref, out_ref.at[pl.ds(my_pos * m_per, m_per), :], own_sem
        )
        own_cp.start()

        for h in range(1, HOPS):
            for j in (0, 1):
                for d in ("cw", "ccw"):
                    mk_in(d, h - 1, j).wait_recv()
                    if j in subs(d, h):
                        if h >= 2:
                            out_rd[(d, h - 2, j)].wait_send()
                        out_rd[(d, h, j)] = mk_out(d, h, j)
                        out_rd[(d, h, j)].start()

        mk_in("cw", HOPS - 1, 0).wait_recv()
        mk_in("ccw", HOPS - 1, 1).wait_recv()

        for d in ("cw", "ccw"):
            for j in (0, 1):
                if j not in subs(d, HOPS - 1):
                    out_rd[(d, HOPS - 3, j)].wait_send()
            for j in subs(d, HOPS - 2):
                out_rd[(d, HOPS - 2, j)].wait_send()
            for j in subs(d, HOPS - 1):
                out_rd[(d, HOPS - 1, j)].wait_send()
        own_cp.wait()

    return pl.pallas_call(
        body,
        out_shape=jax.ShapeDtypeStruct((m_total, n), jnp.bfloat16),
        in_specs=[
            pl.BlockSpec(memory_space=pltpu.VMEM),
            pl.BlockSpec(memory_space=pltpu.VMEM),
        ],
        out_specs=pl.BlockSpec(memory_space=pl.ANY),
        scratch_shapes=[
            pltpu.VMEM((m_per, n), jnp.bfloat16),
            pltpu.SemaphoreType.DMA((2, 2)),
            pltpu.SemaphoreType.DMA((2, 2)),
            pltpu.SemaphoreType.DMA((2, 2)),
            pltpu.SemaphoreType.DMA((2, 2)),
            pltpu.SemaphoreType.DMA,
        ],
        compiler_params=pltpu.CompilerParams(
            collective_id=0, vmem_limit_bytes=100 * 1024 * 1024
        ),
    )(A, B)


# device time: 468204 ns/iter; 3.2744x vs baseline; 1.6614x over previous
import jax
import jax.numpy as jnp
from jax import lax
from jax.experimental import pallas as pl
from jax.experimental.pallas import tpu as pltpu

N_DEV = 16
HOPS = N_DEV // 2

RING = [0, 1, 5, 9, 13, 14, 10, 6, 2, 3, 7, 11, 15, 12, 8, 4]
INDEX_OF = [RING.index(p) for p in range(N_DEV)]


def kernel(A, B):
    m_per, k = A.shape
    _, n = B.shape
    m_total = N_DEV * m_per
    half = m_per // 2

    def body(a_ref, b_ref, out_ref, cw_ref, ccw_ref, stage_ref,
             cw_send, cw_recv, ccw_send, ccw_recv, stage_sems):
        my_pos = lax.axis_index("i")

        def lookup(table, idx):
            v = jnp.int32(table[0])
            for p in range(1, N_DEV):
                v = jnp.where(idx == p, jnp.int32(table[p]), v)
            return v

        my_ring = lookup(INDEX_OF, my_pos)
        right = lookup(RING, lax.rem(my_ring + 1, N_DEV))
        left = lookup(RING, lax.rem(my_ring - 1 + N_DEV, N_DEV))

        def arrive_origin(d, a):
            if d == "cw":
                return lookup(RING, lax.rem(my_ring - (a + 1) + N_DEV, N_DEV))
            return lookup(RING, lax.rem(my_ring + (a + 1), N_DEV))

        def rows(d):
            return pl.ds(0, half) if d == "cw" else pl.ds(half, half)

        def mk_out(d, h):
            ref, send, recv, tgt = (
                (cw_ref, cw_send, cw_recv, right) if d == "cw"
                else (ccw_ref, ccw_send, ccw_recv, left)
            )
            s, r = h % 2, (h + 1) % 2
            if h < HOPS - 1:
                src, dst = ref.at[s], ref.at[r]
            else:
                src, dst = ref.at[s, rows(d), :], ref.at[r, rows(d), :]
            return pltpu.make_async_remote_copy(
                src_ref=src, dst_ref=dst,
                send_sem=send.at[s], recv_sem=recv.at[r],
                device_id=(tgt,), device_id_type=pl.DeviceIdType.MESH,
            )

        def mk_in(d, a):
            ref, send, recv, tgt = (
                (cw_ref, cw_send, cw_recv, right) if d == "cw"
                else (ccw_ref, ccw_send, ccw_recv, left)
            )
            r = (a + 1) % 2
            dst = ref.at[r] if a < HOPS - 1 else ref.at[r, rows(d), :]
            return pltpu.make_async_remote_copy(
                src_ref=dst, dst_ref=dst,
                send_sem=send.at[a % 2], recv_sem=recv.at[r],
                device_id=(tgt,), device_id_type=pl.DeviceIdType.MESH,
            )

        barrier_sem = pltpu.get_barrier_semaphore()
        for nbr in (left, right):
            pl.semaphore_signal(
                barrier_sem, inc=1,
                device_id=(nbr,), device_id_type=pl.DeviceIdType.MESH,
            )

        cw_ref[0] = a_ref[...]
        ccw_ref[0] = a_ref[...]
        b = b_ref[...]

        pl.semaphore_wait(barrier_sem, 2)

        rd = {}
        for d in ("cw", "ccw"):
            rd[(d, 0)] = mk_out(d, 0)
            rd[(d, 0)].start()

        pending = [None, None]

        def emit_tile(a_val, origin, row_off, stage_idx):
            c = jnp.dot(a_val, b, preferred_element_type=jnp.float32)
            if pending[stage_idx] is not None:
                pending[stage_idx].wait()
            stage_ref[stage_idx] = c.astype(jnp.bfloat16)
            cp = pltpu.make_async_copy(
                stage_ref.at[stage_idx],
                out_ref.at[pl.ds(origin * m_per + row_off, half), :],
                stage_sems.at[stage_idx],
            )
            cp.start()
            pending[stage_idx] = cp

        def emit_chunk(ref, slot, origin, a):
            if a < HOPS - 1:
                tiles = (0, half)
            else:
                tiles = (0,) if ref is cw_ref else (half,)
            for t_i, off in enumerate(tiles):
                emit_tile(ref[slot, pl.ds(off, half), :], origin, off,
                          t_i % 2 if len(tiles) > 1 else 0)

        for t_i, off in enumerate((0, half)):
            emit_tile(cw_ref[0, pl.ds(off, half), :], my_pos, off, t_i)

        for a in range(HOPS):
            for d in ("cw", "ccw"):
                mk_in(d, a).wait_recv()
                if a < HOPS - 1:
                    if a >= 1:
                        rd[(d, a - 1)].wait_send()
                    rd[(d, a + 1)] = mk_out(d, a + 1)
                    rd[(d, a + 1)].start()
            slot = (a + 1) % 2
            for d in ("cw", "ccw"):
                ref = cw_ref if d == "cw" else ccw_ref
                emit_chunk(ref, slot, arrive_origin(d, a), a)

        for d in ("cw", "ccw"):
            rd[(d, HOPS - 2)].wait_send()
            rd[(d, HOPS - 1)].wait_send()
        for cp in pending:
            cp.wait()

    return pl.pallas_call(
        body,
        out_shape=jax.ShapeDtypeStruct((m_total, n), jnp.bfloat16),
        in_specs=[
            pl.BlockSpec(memory_space=pltpu.VMEM),
            pl.BlockSpec(memory_space=pltpu.VMEM),
        ],
        out_specs=pl.BlockSpec(memory_space=pl.ANY),
        scratch_shapes=[
            pltpu.VMEM((2, m_per, k), jnp.bfloat16),
            pltpu.VMEM((2, m_per, k), jnp.bfloat16),
            pltpu.VMEM((2, half, n), jnp.bfloat16),
            pltpu.SemaphoreType.DMA((2,)),
            pltpu.SemaphoreType.DMA((2,)),
            pltpu.SemaphoreType.DMA((2,)),
            pltpu.SemaphoreType.DMA((2,)),
            pltpu.SemaphoreType.DMA((2,)),
        ],
        compiler_params=pltpu.CompilerParams(
            collective_id=0, vmem_limit_bytes=100 * 1024 * 1024
        ),
    )(A.astype(jnp.bfloat16), B.astype(jnp.bfloat16))
